# Initial kernel scaffold; baseline (speedup 1.0000x reference)
import jax
import jax.numpy as jnp
from jax.experimental import pallas as pl


def kernel(x, embed):
    raise NotImplementedError("write your pallas kernel here")


# speedup vs baseline: 1.0776x; 1.0776x over previous
import functools

import jax
import jax.numpy as jnp
from jax import lax
from jax.experimental import pallas as pl
from jax.experimental.pallas import tpu as pltpu
from jax.experimental.pallas import tpu_sc as plsc

_DIM = 32
_K = 8192
_BM = 256


def _argmin_body(x_ref, et_ref, idx_ref):
    xb = x_ref[...]
    et = et_ref[...]
    mm = jnp.dot(xb, et, preferred_element_type=jnp.float32)
    x2 = jnp.sum(xb * xb, axis=1, keepdims=True)
    e2 = jnp.sum(et * et, axis=0, keepdims=True)
    dist = -(x2 - 2.0 * mm + e2)
    half = _K // 2
    d1 = dist[:, :half]
    d2 = dist[:, half:]
    m1 = jnp.max(d1, axis=1, keepdims=True)
    m2 = jnp.max(d2, axis=1, keepdims=True)
    col = lax.broadcasted_iota(jnp.int32, (_BM, half), 1)
    i1 = jnp.min(jnp.where(d1 == m1, col, _K), axis=1)
    i2 = jnp.min(jnp.where(d2 == m2, col, _K), axis=1) + half
    m1b = m1[:, 0].astype(jnp.bfloat16).astype(jnp.float32)
    idx = jnp.where(m2[:, 0] > m1b, i2, i1)
    idx_ref[...] = idx.astype(jnp.int32)


def _nearest_idx(flat, embed_t):
    m = flat.shape[0]
    return pl.pallas_call(
        _argmin_body,
        grid=(m // _BM,),
        in_specs=[
            pl.BlockSpec((_BM, _DIM), lambda i: (i, 0)),
            pl.BlockSpec((_DIM, _K), lambda i: (0, 0)),
        ],
        out_specs=pl.BlockSpec((_BM,), lambda i: (i,)),
        out_shape=jax.ShapeDtypeStruct((m,), jnp.int32),
        compiler_params=pltpu.CompilerParams(
            dimension_semantics=("arbitrary",),
        ),
    )(flat, embed_t)


def _gather_rows(embed, idx_flat):
    info = plsc.get_sparse_core_info()
    nw = info.num_cores * info.num_subcores
    b = idx_flat.shape[0]
    bpw = b // nw
    nchunk = bpw // 128
    mesh = plsc.VectorSubcoreMesh(core_axis_name="c", subcore_axis_name="s")

    @functools.partial(
        pl.kernel,
        mesh=mesh,
        out_type=jax.ShapeDtypeStruct((b, _DIM), jnp.float32),
        scratch_types=[
            pltpu.VMEM((nchunk, 128), jnp.int32),
            pltpu.VMEM((bpw, _DIM), jnp.float32),
            pltpu.SemaphoreType.DMA,
        ],
        compiler_params=pltpu.CompilerParams(use_tc_tiling_on_sc=False),
    )
    def gk(table_hbm, idx_hbm, out_hbm, idx_v, rows_v, sem):
        wid = lax.axis_index("c") * info.num_subcores + lax.axis_index("s")
        base = wid * bpw
        for j in range(nchunk):
            pltpu.sync_copy(idx_hbm.at[pl.ds(base + j * 128, 128)],
                            idx_v.at[j])
        copies = [
            pltpu.async_copy(table_hbm.at[idx_v.at[j]],
                             rows_v.at[pl.ds(j * 128, 128)], sem)
            for j in range(nchunk)
        ]
        for c in copies:
            c.wait()
        pltpu.sync_copy(rows_v, out_hbm.at[pl.ds(base, bpw)])

    return gk(embed, idx_flat)


def kernel(x, embed):
    shape = x.shape
    flat = x.reshape(-1, shape[-1])
    idx_flat = _nearest_idx(flat, embed.T)
    quant = _gather_rows(embed, idx_flat)
    return (quant.reshape(shape), idx_flat.reshape(shape[:-1]))


# speedup vs baseline: 1.3593x; 1.2615x over previous
import functools

import jax
import jax.numpy as jnp
from jax import lax
from jax.experimental import pallas as pl
from jax.experimental.pallas import tpu as pltpu
from jax.experimental.pallas import tpu_sc as plsc

_DIM = 32
_K = 8192
_BM = 256


def _half_argmin(v):
    nch = v.shape[1] // 128
    best = v[:, 0:128]
    bj = jnp.zeros((_BM, 128), jnp.int32)
    for j in range(1, nch):
        c = v[:, j * 128:(j + 1) * 128]
        m = c < best
        best = jnp.where(m, c, best)
        bj = jnp.where(m, j, bj)
    rowmin = jnp.min(best, axis=1, keepdims=True)
    lane = lax.broadcasted_iota(jnp.int32, (_BM, 128), 1)
    cand = jnp.where(best == rowmin, bj * 128 + lane, _K)
    return rowmin[:, 0], jnp.min(cand, axis=1)


def _argmin_body(x_ref, et_ref, idx_ref):
    xb = x_ref[...]
    et = et_ref[...]
    mm = jnp.dot(xb, et, preferred_element_type=jnp.float32)
    x2 = jnp.sum(xb * xb, axis=1, keepdims=True)
    e2 = jnp.sum(et * et, axis=0, keepdims=True)
    v = (x2 - 2.0 * mm) + e2
    half = _K // 2
    m1, i1 = _half_argmin(v[:, :half])
    m2, i2 = _half_argmin(v[:, half:])
    m1b = m1.astype(jnp.bfloat16).astype(jnp.float32)
    idx = jnp.where(m2 < m1b, i2 + half, i1)
    idx_ref[...] = idx.astype(jnp.int32)


def _nearest_idx(flat, embed_t):
    m = flat.shape[0]
    return pl.pallas_call(
        _argmin_body,
        grid=(m // _BM,),
        in_specs=[
            pl.BlockSpec((_BM, _DIM), lambda i: (i, 0)),
            pl.BlockSpec((_DIM, _K), lambda i: (0, 0)),
        ],
        out_specs=pl.BlockSpec((_BM,), lambda i: (i,)),
        out_shape=jax.ShapeDtypeStruct((m,), jnp.int32),
        compiler_params=pltpu.CompilerParams(
            dimension_semantics=("arbitrary",),
        ),
    )(flat, embed_t)


def _gather_rows(embed, idx_flat):
    info = plsc.get_sparse_core_info()
    nw = info.num_cores * info.num_subcores
    b = idx_flat.shape[0]
    bpw = b // nw
    nchunk = bpw // 128
    mesh = plsc.VectorSubcoreMesh(core_axis_name="c", subcore_axis_name="s")

    @functools.partial(
        pl.kernel,
        mesh=mesh,
        out_type=jax.ShapeDtypeStruct((b, _DIM), jnp.float32),
        scratch_types=[
            pltpu.VMEM((nchunk, 128), jnp.int32),
            pltpu.VMEM((bpw, _DIM), jnp.float32),
            pltpu.SemaphoreType.DMA,
        ],
        compiler_params=pltpu.CompilerParams(use_tc_tiling_on_sc=False),
    )
    def gk(table_hbm, idx_hbm, out_hbm, idx_v, rows_v, sem):
        wid = lax.axis_index("c") * info.num_subcores + lax.axis_index("s")
        base = wid * bpw
        for j in range(nchunk):
            pltpu.sync_copy(idx_hbm.at[pl.ds(base + j * 128, 128)],
                            idx_v.at[j])
        copies = [
            pltpu.async_copy(table_hbm.at[idx_v.at[j]],
                             rows_v.at[pl.ds(j * 128, 128)], sem)
            for j in range(nchunk)
        ]
        for c in copies:
            c.wait()
        pltpu.sync_copy(rows_v, out_hbm.at[pl.ds(base, bpw)])

    return gk(embed, idx_flat)


def kernel(x, embed):
    shape = x.shape
    flat = x.reshape(-1, shape[-1])
    idx_flat = _nearest_idx(flat, embed.T)
    quant = _gather_rows(embed, idx_flat)
    return (quant.reshape(shape), idx_flat.reshape(shape[:-1]))
